# speedup vs baseline: 1.0364x; 1.0364x over previous
import functools

import jax
import jax.numpy as jnp
import numpy as np
from jax.experimental import pallas as pl
from jax.experimental.pallas import tpu as pltpu

N = 10000
E = 160000
D = 256
H = 16
C = D // H
ED = 16
HID = 256

ROWS = 400


def _ln(x, g, b, eps=1e-5):
    m = x.mean(-1, keepdims=True)
    v = ((x - m) ** 2).mean(-1, keepdims=True)
    return (x - m) / jnp.sqrt(v + eps) * g + b


def _pre_body(x_ref, g_ref, b_ref, wq_ref, bq_ref, wk_ref, bk_ref,
              wv_ref, bv_ref, ws_ref, bs_ref,
              q_ref, k_ref, v_ref, xr_ref):
    xn = _ln(x_ref[...], g_ref[...], b_ref[...])
    q_ref[...] = jnp.dot(xn, wq_ref[...], preferred_element_type=jnp.float32) + bq_ref[...]
    k_ref[...] = jnp.dot(xn, wk_ref[...], preferred_element_type=jnp.float32) + bk_ref[...]
    v_ref[...] = jnp.dot(xn, wv_ref[...], preferred_element_type=jnp.float32) + bv_ref[...]
    xr_ref[...] = jnp.dot(xn, ws_ref[...], preferred_element_type=jnp.float32) + bs_ref[...]


def _pre(x, g, b, wq, bq, wk, bk, wv, bv, ws, bs):
    row = pl.BlockSpec((ROWS, D), lambda i: (i, 0))
    full = pl.BlockSpec((D, D), lambda i: (0, 0))
    vec = pl.BlockSpec((D,), lambda i: (0,))
    return pl.pallas_call(
        _pre_body,
        grid=(N // ROWS,),
        in_specs=[row, vec, vec, full, vec, full, vec, full, vec, full, vec],
        out_specs=[row, row, row, row],
        out_shape=[jax.ShapeDtypeStruct((N, D), jnp.float32)] * 4,
    )(x, g, b, wq, bq, wk, bk, wv, bv, ws, bs)


EROWS = 4000


def _eemb_body(ea_ref, we_ref, be_ref, o_ref):
    o_ref[...] = jnp.dot(ea_ref[...], we_ref[...], preferred_element_type=jnp.float32) + be_ref[...]


def _eemb(ea, we, be):
    return pl.pallas_call(
        _eemb_body,
        grid=(E // EROWS,),
        in_specs=[pl.BlockSpec((EROWS, ED), lambda i: (i, 0)),
                  pl.BlockSpec((ED, D), lambda i: (0, 0)),
                  pl.BlockSpec((D,), lambda i: (0,))],
        out_specs=pl.BlockSpec((EROWS, D), lambda i: (i, 0)),
        out_shape=jax.ShapeDtypeStruct((E, D), jnp.float32),
    )(ea, we, be)


def _tail_body(agg_ref, xr_ref, xs_ref, wp_ref, bp_ref, g_ref, b_ref,
               w1_ref, b1_ref, w2_ref, b2_ref, o_ref):
    out = jnp.dot(agg_ref[...] + xr_ref[...], wp_ref[...],
                  preferred_element_type=jnp.float32) + bp_ref[...] + xs_ref[...]
    h = _ln(out, g_ref[...], b_ref[...])
    h = jnp.dot(h, w1_ref[...], preferred_element_type=jnp.float32) + b1_ref[...]
    h = h * jax.nn.sigmoid(h)
    h = jnp.dot(h, w2_ref[...], preferred_element_type=jnp.float32) + b2_ref[...]
    o_ref[...] = h + out


def _tail(agg, xr, xs, wp, bp, g, b, w1, b1, w2, b2):
    row = pl.BlockSpec((ROWS, D), lambda i: (i, 0))
    return pl.pallas_call(
        _tail_body,
        grid=(N // ROWS,),
        in_specs=[row, row, row,
                  pl.BlockSpec((D, D), lambda i: (0, 0)),
                  pl.BlockSpec((D,), lambda i: (0,)),
                  pl.BlockSpec((D,), lambda i: (0,)),
                  pl.BlockSpec((D,), lambda i: (0,)),
                  pl.BlockSpec((D, HID), lambda i: (0, 0)),
                  pl.BlockSpec((HID,), lambda i: (0,)),
                  pl.BlockSpec((HID, D), lambda i: (0, 0)),
                  pl.BlockSpec((D,), lambda i: (0,))],
        out_specs=row,
        out_shape=jax.ShapeDtypeStruct((N, D), jnp.float32),
    )(agg, xr, xs, wp, bp, g, b, w1, b1, w2, b2)


def kernel(x, edge_index, edge_attr, ln1_g, ln1_b, Wq, bq, Wk, bk, Wv, bv,
           Ws, bs, We, be, Wp, bp, mlp_ln_g, mlp_ln_b, W1, b1, W2, b2):
    q, k, v, x_r = _pre(x, ln1_g, ln1_b, Wq, bq, Wk, bk, Wv, bv, Ws, bs)
    eemb = _eemb(edge_attr, We, be)

    src = edge_index[0].astype(jnp.int32)
    dst = edge_index[1].astype(jnp.int32)

    qh = q.reshape(N, H, C)
    kh = k.reshape(N, H, C)
    vh = v.reshape(N, H, C)
    q_i = jnp.take(qh, dst, axis=0)
    k_j = jnp.take(kh, src, axis=0) + eemb.reshape(E, H, C)
    v_j = jnp.take(vh, src, axis=0)
    alpha = (q_i * k_j).sum(-1) / np.sqrt(C)
    ex = jnp.exp(alpha)
    den = jax.ops.segment_sum(ex, dst, num_segments=N)
    msg = v_j * ex[..., None]
    acc = jax.ops.segment_sum(msg, dst, num_segments=N)
    agg = (acc / (den[..., None] + 1e-16)).reshape(N, D)

    return _tail(agg, x_r, x, Wp, bp, mlp_ln_g, mlp_ln_b, W1, b1, W2, b2)


# speedup vs baseline: 18.0964x; 17.4609x over previous
import functools

import jax
import jax.numpy as jnp
from jax import lax
from jax.experimental import pallas as pl
from jax.experimental.pallas import tpu as pltpu
import jax.experimental.pallas.tpu_sc as plsc

N = 10000
E = 160000
D = 256
H = 16
C = D // H
ED = 16
HID = 256

ROWS = 400
EROWS = 4000

NC = 2
NS = 16
NW = NC * NS

B1 = 40
EW1 = E // NW
B2 = 80
EW2 = E // NS
RB = 40
NRCH = N // RB
DH = D // 2
DW = 128


def _ln(x, g, b, eps=1e-5):
    m = x.mean(-1, keepdims=True)
    v = ((x - m) ** 2).mean(-1, keepdims=True)
    return (x - m) / jnp.sqrt(v + eps) * g + b


def _pre_body(x_ref, g_ref, b_ref, wq_ref, bq_ref, wk_ref, bk_ref,
              wv_ref, bv_ref, ws_ref, bs_ref,
              q_ref, k_ref, vlo_ref, vhi_ref, xr_ref):
    xn = _ln(x_ref[...], g_ref[...], b_ref[...])
    q_ref[...] = jnp.dot(xn, wq_ref[...], preferred_element_type=jnp.float32) + bq_ref[...]
    k_ref[...] = jnp.dot(xn, wk_ref[...], preferred_element_type=jnp.float32) + bk_ref[...]
    v = jnp.dot(xn, wv_ref[...], preferred_element_type=jnp.float32) + bv_ref[...]
    vlo_ref[...] = v[:, :DH]
    vhi_ref[...] = v[:, DH:]
    xr_ref[...] = jnp.dot(xn, ws_ref[...], preferred_element_type=jnp.float32) + bs_ref[...]


def _pre(x, g, b, wq, bq, wk, bk, wv, bv, ws, bs):
    row = pl.BlockSpec((ROWS, D), lambda i: (i, 0))
    half = pl.BlockSpec((ROWS, DH), lambda i: (i, 0))
    full = pl.BlockSpec((D, D), lambda i: (0, 0))
    vec = pl.BlockSpec((D,), lambda i: (0,))
    return pl.pallas_call(
        _pre_body,
        grid=(N // ROWS,),
        in_specs=[row, vec, vec, full, vec, full, vec, full, vec, full, vec],
        out_specs=[row, row, half, half, row],
        out_shape=[jax.ShapeDtypeStruct((N, D), jnp.float32),
                   jax.ShapeDtypeStruct((N, D), jnp.float32),
                   jax.ShapeDtypeStruct((N, DH), jnp.float32),
                   jax.ShapeDtypeStruct((N, DH), jnp.float32),
                   jax.ShapeDtypeStruct((N, D), jnp.float32)],
    )(x, g, b, wq, bq, wk, bk, wv, bv, ws, bs)


def _eemb_body(ea_ref, we_ref, be_ref, o_ref):
    o_ref[...] = jnp.dot(ea_ref[...], we_ref[...], preferred_element_type=jnp.float32) + be_ref[...]


def _eemb(ea, we, be):
    return pl.pallas_call(
        _eemb_body,
        grid=(E // EROWS,),
        in_specs=[pl.BlockSpec((EROWS, ED), lambda i: (i, 0)),
                  pl.BlockSpec((ED, D), lambda i: (0, 0)),
                  pl.BlockSpec((D,), lambda i: (0,))],
        out_specs=pl.BlockSpec((EROWS, D), lambda i: (i, 0)),
        out_shape=jax.ShapeDtypeStruct((E, D), jnp.float32),
    )(ea, we, be)


def _tail_body(agg_ref, xr_ref, xs_ref, wp_ref, bp_ref, g_ref, b_ref,
               w1_ref, b1_ref, w2_ref, b2_ref, o_ref):
    out = jnp.dot(agg_ref[...] + xr_ref[...], wp_ref[...],
                  preferred_element_type=jnp.float32) + bp_ref[...] + xs_ref[...]
    h = _ln(out, g_ref[...], b_ref[...])
    h = jnp.dot(h, w1_ref[...], preferred_element_type=jnp.float32) + b1_ref[...]
    h = h * jax.nn.sigmoid(h)
    h = jnp.dot(h, w2_ref[...], preferred_element_type=jnp.float32) + b2_ref[...]
    o_ref[...] = h + out


def _tail(agg, xr, xs, wp, bp, g, b, w1, b1, w2, b2):
    row = pl.BlockSpec((ROWS, D), lambda i: (i, 0))
    return pl.pallas_call(
        _tail_body,
        grid=(N // ROWS,),
        in_specs=[row, row, row,
                  pl.BlockSpec((D, D), lambda i: (0, 0)),
                  pl.BlockSpec((D,), lambda i: (0,)),
                  pl.BlockSpec((D,), lambda i: (0,)),
                  pl.BlockSpec((D,), lambda i: (0,)),
                  pl.BlockSpec((D, HID), lambda i: (0, 0)),
                  pl.BlockSpec((HID,), lambda i: (0,)),
                  pl.BlockSpec((HID, D), lambda i: (0, 0)),
                  pl.BlockSpec((D,), lambda i: (0,))],
        out_specs=row,
        out_shape=jax.ShapeDtypeStruct((N, D), jnp.float32),
    )(agg, xr, xs, wp, bp, g, b, w1, b1, w2, b2)


def _sc_mesh():
    return plsc.VectorSubcoreMesh(core_axis_name="c", subcore_axis_name="s",
                                  num_cores=NC, num_subcores=NS)


def _sc_attn(qp, kp, eemb, src, dst, zden):
    @functools.partial(
        pl.kernel,
        out_type=[jax.ShapeDtypeStruct((E, H), jnp.float32),
                  jax.ShapeDtypeStruct((NC * N, DW), jnp.float32)],
        mesh=_sc_mesh(),
        scratch_types=[
            pltpu.VMEM((B1,), jnp.int32),
            pltpu.VMEM((B1,), jnp.int32),
            pltpu.VMEM((B1, D), jnp.float32),
            pltpu.VMEM((B1, D), jnp.float32),
            pltpu.VMEM((B1, D), jnp.float32),
            pltpu.VMEM((B1, H), jnp.float32),
            pltpu.VMEM((B1, DW), jnp.float32),
            pltpu.VMEM_SHARED((N, DW), jnp.float32),
            pltpu.SemaphoreType.DMA,
        ],
    )
    def k(q_hbm, k_hbm, e_hbm, src_hbm, dst_hbm, zden_hbm, ex_hbm, den_hbm,
          srcv, dstv, qbuf, kbuf, ebuf, exbuf, exwide, den_sp, sem):
        cid = lax.axis_index("c")
        sid = lax.axis_index("s")
        wid = sid * NC + cid
        base0 = wid * EW1

        @pl.when(sid == 0)
        def _():
            pltpu.sync_copy(zden_hbm, den_sp)

        def zrow(e, carry):
            for j in range(DW // H):
                exwide[e, pl.ds(j * H, H)] = jnp.zeros((H,), jnp.float32)
            return carry

        lax.fori_loop(0, B1, zrow, 0)
        plsc.subcore_barrier()

        def chunk(i, carry):
            base = base0 + i * B1
            pltpu.sync_copy(src_hbm.at[pl.ds(base, B1)], srcv)
            pltpu.sync_copy(dst_hbm.at[pl.ds(base, B1)], dstv)
            cq = pltpu.async_copy(q_hbm.at[dstv], qbuf, sem)
            ck = pltpu.async_copy(k_hbm.at[srcv], kbuf, sem)
            ce = pltpu.async_copy(e_hbm.at[pl.ds(base, B1)], ebuf, sem)
            cq.wait()
            ck.wait()
            ce.wait()

            def edge(e, ecarry):
                acc = jnp.zeros((H,), jnp.float32)
                for c in range(C):
                    sl = pl.ds(c * H, H)
                    acc = acc + qbuf[e, sl] * (kbuf[e, sl] + ebuf[e, sl])
                ex = jnp.exp(acc)
                exbuf[e, :] = ex
                exwide[e, pl.ds(0, H)] = ex
                return ecarry

            lax.fori_loop(0, B1, edge, 0)
            pltpu.sync_copy(exbuf, ex_hbm.at[pl.ds(base, B1)])
            pltpu.sync_copy(exwide, den_sp.at[dstv], add=True)
            return carry

        lax.fori_loop(0, EW1 // B1, chunk, 0)

        plsc.subcore_barrier()

        def drows(t, carry):
            ch = t * NS + sid

            @pl.when(ch < NRCH)
            def _():
                r0 = ch * RB
                pltpu.sync_copy(den_sp.at[pl.ds(r0, RB)],
                                den_hbm.at[pl.ds(cid * N + r0, RB)])

            return carry

        lax.fori_loop(0, (NRCH + NS - 1) // NS, drows, 0)

    return k(qp, kp, eemb, src, dst, zden)


def _sc_agg(vlo, vhi, exv, src, dst, den2, zacc):
    @functools.partial(
        pl.kernel,
        out_type=jax.ShapeDtypeStruct((NC * N, DH), jnp.float32),
        mesh=_sc_mesh(),
        scratch_types=[
            pltpu.VMEM((B2,), jnp.int32),
            pltpu.VMEM((B2,), jnp.int32),
            pltpu.VMEM((B2, DH), jnp.float32),
            pltpu.VMEM((B2, H), jnp.float32),
            pltpu.VMEM((B2, DH), jnp.float32),
            pltpu.VMEM((RB, DH), jnp.float32),
            pltpu.VMEM((RB, DW), jnp.float32),
            pltpu.VMEM((RB, DW), jnp.float32),
            pltpu.VMEM_SHARED((N, DH), jnp.float32),
            pltpu.SemaphoreType.DMA,
        ],
    )
    def k(vlo_hbm, vhi_hbm, ex_hbm, src_hbm, dst_hbm, den_hbm, zacc_hbm,
          out_hbm, srcv, dstv, vbuf, exbuf, msgbuf, abuf, dbuf, dbuf2,
          acc_sp, sem):
        cid = lax.axis_index("c")
        sid = lax.axis_index("s")

        @pl.when(sid == 0)
        def _():
            pltpu.sync_copy(zacc_hbm, acc_sp)

        plsc.subcore_barrier()

        base0 = sid * EW2

        def chunk(i, carry):
            base = base0 + i * B2
            pltpu.sync_copy(src_hbm.at[pl.ds(base, B2)], srcv)
            pltpu.sync_copy(dst_hbm.at[pl.ds(base, B2)], dstv)
            pltpu.sync_copy(ex_hbm.at[pl.ds(base, B2)], exbuf)

            @pl.when(cid == 0)
            def _():
                pltpu.async_copy(vlo_hbm.at[srcv], vbuf, sem).wait()

            @pl.when(cid == 1)
            def _():
                pltpu.async_copy(vhi_hbm.at[srcv], vbuf, sem).wait()

            def edge(e, ecarry):
                ev = exbuf[e, :]
                for j in range(DH // H):
                    sl = pl.ds(j * H, H)
                    msgbuf[e, sl] = vbuf[e, sl] * ev
                return ecarry

            lax.fori_loop(0, B2, edge, 0)
            pltpu.sync_copy(msgbuf, acc_sp.at[dstv], add=True)
            return carry

        lax.fori_loop(0, EW2 // B2, chunk, 0)

        plsc.subcore_barrier()

        def rows(t, carry):
            ch = t * NS + sid

            @pl.when(ch < NRCH)
            def _():
                r0 = ch * RB
                pltpu.sync_copy(acc_sp.at[pl.ds(r0, RB)], abuf)
                pltpu.sync_copy(den_hbm.at[pl.ds(r0, RB)], dbuf)
                pltpu.sync_copy(den_hbm.at[pl.ds(N + r0, RB)], dbuf2)

                def row(r, rcarry):
                    dsl = pl.ds(0, H)
                    rec = 1.0 / (dbuf[r, dsl] + dbuf2[r, dsl] + 1e-16)
                    for j in range(DH // H):
                        sl = pl.ds(j * H, H)
                        abuf[r, sl] = abuf[r, sl] * rec
                    return rcarry

                lax.fori_loop(0, RB, row, 0)
                pltpu.sync_copy(abuf, out_hbm.at[pl.ds(cid * N + r0, RB)])

            return carry

        lax.fori_loop(0, (NRCH + NS - 1) // NS, rows, 0)

    return k(vlo, vhi, exv, src, dst, den2, zacc)


def _perm_cols(w):
    shp = w.shape[:-1]
    return w.reshape(*shp, H, C).swapaxes(-1, -2).reshape(*shp, D)


def kernel(x, edge_index, edge_attr, ln1_g, ln1_b, Wq, bq, Wk, bk, Wv, bv,
           Ws, bs, We, be, Wp, bp, mlp_ln_g, mlp_ln_b, W1, b1, W2, b2):
    scale = 1.0 / (C ** 0.5)
    wq_p = _perm_cols(Wq) * scale
    bq_p = _perm_cols(bq) * scale
    wk_p = _perm_cols(Wk)
    bk_p = _perm_cols(bk)
    wv_p = _perm_cols(Wv)
    bv_p = _perm_cols(bv)
    ws_p = _perm_cols(Ws)
    bs_p = _perm_cols(bs)
    we_p = _perm_cols(We)
    be_p = _perm_cols(be)
    wp_p = Wp.reshape(H, C, D).swapaxes(0, 1).reshape(D, D)

    q, kk, vlo, vhi, xr = _pre(x, ln1_g, ln1_b, wq_p, bq_p, wk_p, bk_p,
                               wv_p, bv_p, ws_p, bs_p)
    eemb = _eemb(edge_attr, we_p, be_p)

    src = edge_index[0].astype(jnp.int32)
    dst = edge_index[1].astype(jnp.int32)

    zden = jnp.zeros((N, DW), jnp.float32)
    exv, den2 = _sc_attn(q, kk, eemb, src, dst, zden)

    zacc = jnp.zeros((N, DH), jnp.float32)
    out2 = _sc_agg(vlo, vhi, exv, src, dst, den2, zacc)
    agg = out2.reshape(NC, N, DH).transpose(1, 0, 2).reshape(N, D)

    return _tail(agg, xr, x, wp_p, bp, mlp_ln_g, mlp_ln_b, W1, b1, W2, b2)


# speedup vs baseline: 32.5504x; 1.7987x over previous
import functools

import jax
import jax.numpy as jnp
from jax import lax
from jax.experimental import pallas as pl
from jax.experimental.pallas import tpu as pltpu
import jax.experimental.pallas.tpu_sc as plsc

N = 10000
E = 160000
D = 256
H = 16
C = D // H
ED = 16
HID = 256

ROWS = 400
EROWS = 4000

NC = 2
NS = 16
NW = NC * NS

B1 = 40
EW1 = E // NW
NCH1 = EW1 // B1
B2 = 40
EW2 = E // NS
NCH2 = EW2 // B2
RB = 40
NRCH = N // RB
RB2 = 40
NRCH2 = N // RB2
DH = D // 2
DW = 128


def _ln(x, g, b, eps=1e-5):
    m = x.mean(-1, keepdims=True)
    v = ((x - m) ** 2).mean(-1, keepdims=True)
    return (x - m) / jnp.sqrt(v + eps) * g + b


def _pre_body(x_ref, g_ref, b_ref, wq_ref, bq_ref, wk_ref, bk_ref,
              wv_ref, bv_ref, ws_ref, bs_ref,
              q_ref, k_ref, vlo_ref, vhi_ref, xr_ref):
    xn = _ln(x_ref[...], g_ref[...], b_ref[...])
    q_ref[...] = jnp.dot(xn, wq_ref[...], preferred_element_type=jnp.float32) + bq_ref[...]
    k_ref[...] = jnp.dot(xn, wk_ref[...], preferred_element_type=jnp.float32) + bk_ref[...]
    v = jnp.dot(xn, wv_ref[...], preferred_element_type=jnp.float32) + bv_ref[...]
    vlo_ref[...] = v[:, :DH]
    vhi_ref[...] = v[:, DH:]
    xr_ref[...] = jnp.dot(xn, ws_ref[...], preferred_element_type=jnp.float32) + bs_ref[...]


def _pre(x, g, b, wq, bq, wk, bk, wv, bv, ws, bs):
    row = pl.BlockSpec((ROWS, D), lambda i: (i, 0))
    half = pl.BlockSpec((ROWS, DH), lambda i: (i, 0))
    full = pl.BlockSpec((D, D), lambda i: (0, 0))
    vec = pl.BlockSpec((D,), lambda i: (0,))
    return pl.pallas_call(
        _pre_body,
        grid=(N // ROWS,),
        in_specs=[row, vec, vec, full, vec, full, vec, full, vec, full, vec],
        out_specs=[row, row, half, half, row],
        out_shape=[jax.ShapeDtypeStruct((N, D), jnp.float32),
                   jax.ShapeDtypeStruct((N, D), jnp.float32),
                   jax.ShapeDtypeStruct((N, DH), jnp.float32),
                   jax.ShapeDtypeStruct((N, DH), jnp.float32),
                   jax.ShapeDtypeStruct((N, D), jnp.float32)],
    )(x, g, b, wq, bq, wk, bk, wv, bv, ws, bs)


def _eemb_body(ea_ref, we_ref, be_ref, o_ref):
    o_ref[...] = jnp.dot(ea_ref[...], we_ref[...], preferred_element_type=jnp.float32) + be_ref[...]


def _eemb(ea, we, be):
    return pl.pallas_call(
        _eemb_body,
        grid=(E // EROWS,),
        in_specs=[pl.BlockSpec((EROWS, ED), lambda i: (i, 0)),
                  pl.BlockSpec((ED, D), lambda i: (0, 0)),
                  pl.BlockSpec((D,), lambda i: (0,))],
        out_specs=pl.BlockSpec((EROWS, D), lambda i: (i, 0)),
        out_shape=jax.ShapeDtypeStruct((E, D), jnp.float32),
    )(ea, we, be)


def _tail_body(alo_ref, ahi_ref, xr_ref, xs_ref, wp_ref, bp_ref, g_ref, b_ref,
               w1_ref, b1_ref, w2_ref, b2_ref, o_ref):
    agg = jnp.concatenate([alo_ref[...], ahi_ref[...]], axis=1)
    out = jnp.dot(agg + xr_ref[...], wp_ref[...],
                  preferred_element_type=jnp.float32) + bp_ref[...] + xs_ref[...]
    h = _ln(out, g_ref[...], b_ref[...])
    h = jnp.dot(h, w1_ref[...], preferred_element_type=jnp.float32) + b1_ref[...]
    h = h * jax.nn.sigmoid(h)
    h = jnp.dot(h, w2_ref[...], preferred_element_type=jnp.float32) + b2_ref[...]
    o_ref[...] = h + out


def _tail(out2, xr, xs, wp, bp, g, b, w1, b1, w2, b2):
    row = pl.BlockSpec((ROWS, D), lambda i: (i, 0))
    nr = N // ROWS
    halfa = pl.BlockSpec((ROWS, DH), lambda i: (i, 0))
    halfb = pl.BlockSpec((ROWS, DH), lambda i: (i + nr, 0))
    return pl.pallas_call(
        _tail_body,
        grid=(nr,),
        in_specs=[halfa, halfb, row, row,
                  pl.BlockSpec((D, D), lambda i: (0, 0)),
                  pl.BlockSpec((D,), lambda i: (0,)),
                  pl.BlockSpec((D,), lambda i: (0,)),
                  pl.BlockSpec((D,), lambda i: (0,)),
                  pl.BlockSpec((D, HID), lambda i: (0, 0)),
                  pl.BlockSpec((HID,), lambda i: (0,)),
                  pl.BlockSpec((HID, D), lambda i: (0, 0)),
                  pl.BlockSpec((D,), lambda i: (0,))],
        out_specs=row,
        out_shape=jax.ShapeDtypeStruct((N, D), jnp.float32),
    )(out2, out2, xr, xs, wp, bp, g, b, w1, b1, w2, b2)


def _sc_mesh():
    return plsc.VectorSubcoreMesh(core_axis_name="c", subcore_axis_name="s",
                                  num_cores=NC, num_subcores=NS)


def _sc_attn(qp, kp, eemb, src, dst):
    @functools.partial(
        pl.kernel,
        out_type=jax.ShapeDtypeStruct((E, H), jnp.float32),
        mesh=_sc_mesh(),
        scratch_types=[
            pltpu.VMEM((EW1,), jnp.int32),
            pltpu.VMEM((EW1,), jnp.int32),
            [pltpu.VMEM((B1, D), jnp.float32)] * 2,
            [pltpu.VMEM((B1, D), jnp.float32)] * 2,
            [pltpu.VMEM((B1, D), jnp.float32)] * 2,
            [pltpu.VMEM((B1, H), jnp.float32)] * 2,
            [pltpu.SemaphoreType.DMA] * 2,
            [pltpu.SemaphoreType.DMA] * 2,
        ],
    )
    def k(q_hbm, k_hbm, e_hbm, src_hbm, dst_hbm, ex_hbm,
          srcall, dstall, qb, kb, eb, exb, semr, semw):
        cid = lax.axis_index("c")
        sid = lax.axis_index("s")
        wid = sid * NC + cid
        base0 = wid * EW1

        pltpu.sync_copy(src_hbm.at[pl.ds(base0, EW1)], srcall)
        pltpu.sync_copy(dst_hbm.at[pl.ds(base0, EW1)], dstall)

        def issue(i, b):
            base = base0 + i * B1
            pltpu.async_copy(q_hbm.at[dstall.at[pl.ds(i * B1, B1)]], qb[b], semr[b])
            pltpu.async_copy(k_hbm.at[srcall.at[pl.ds(i * B1, B1)]], kb[b], semr[b])
            pltpu.async_copy(e_hbm.at[pl.ds(base, B1)], eb[b], semr[b])

        def wait_in(b):
            pltpu.make_async_copy(q_hbm.at[dstall.at[pl.ds(0, B1)]], qb[b], semr[b]).wait()
            pltpu.make_async_copy(k_hbm.at[srcall.at[pl.ds(0, B1)]], kb[b], semr[b]).wait()
            pltpu.make_async_copy(e_hbm.at[pl.ds(0, B1)], eb[b], semr[b]).wait()

        def drain_w(b):
            pltpu.make_async_copy(exb[b], ex_hbm.at[pl.ds(0, B1)], semw[b]).wait()

        def compute(b):
            def edge(e2, ecarry):
                for o in range(2):
                    e = e2 * 2 + o
                    acc = jnp.zeros((H,), jnp.float32)
                    for c in range(C):
                        sl = pl.ds(c * H, H)
                        acc = acc + qb[b][e, sl] * (kb[b][e, sl] + eb[b][e, sl])
                    exb[b][e, :] = jnp.exp(acc)
                return ecarry

            lax.fori_loop(0, B1 // 2, edge, 0)

        def step(i, u):
            b = u % 2
            wait_in(b)

            @pl.when(i >= 2)
            def _():
                drain_w(b)

            compute(b)
            base = base0 + i * B1
            pltpu.async_copy(exb[b], ex_hbm.at[pl.ds(base, B1)], semw[b])
            nxt = i + 2

            @pl.when(nxt < NCH1)
            def _():
                issue(nxt, b)

        issue(0, 0)
        issue(1, 1)

        def gbody(g, carry):
            for u in range(2):
                step(g * 2 + u, u)
            return carry

        lax.fori_loop(0, NCH1 // 2, gbody, 0)

        for t in range(NCH1 - (NCH1 // 2) * 2):
            step((NCH1 // 2) * 2 + t, t)
        drain_w(0)
        drain_w(1)

    return k(qp, kp, eemb, src, dst)


def _sc_den(exv, dst, zden):
    @functools.partial(
        pl.kernel,
        out_type=jax.ShapeDtypeStruct((NC * N, DW), jnp.float32),
        mesh=_sc_mesh(),
        scratch_types=[
            [pltpu.VMEM((B1, H), jnp.float32)] * 2,
            [pltpu.VMEM((B1, DW), jnp.float32)] * 2,
            [pltpu.VMEM((B1,), jnp.int32)] * 4,
            pltpu.VMEM_SHARED((N, DW), jnp.float32),
            [pltpu.SemaphoreType.DMA] * 2,
            [pltpu.SemaphoreType.DMA] * 2,
        ],
    )
    def k(ex_hbm, dst_hbm, zden_hbm, den_hbm,
          exb, exw, dv, den_sp, semr, semw):
        cid = lax.axis_index("c")
        sid = lax.axis_index("s")
        wid = sid * NC + cid
        base0 = wid * EW1

        @pl.when(sid == 0)
        def _():
            pltpu.sync_copy(zden_hbm, den_sp)

        def zrow(e, carry):
            for b in range(2):
                for j in range(DW // H):
                    exw[b][e, pl.ds(j * H, H)] = jnp.zeros((H,), jnp.float32)
            return carry

        lax.fori_loop(0, B1, zrow, 0)
        plsc.subcore_barrier()

        def issue(i, b, d4):
            base = base0 + i * B1
            pltpu.async_copy(ex_hbm.at[pl.ds(base, B1)], exb[b], semr[b])
            pltpu.async_copy(dst_hbm.at[pl.ds(base, B1)], dv[d4], semr[b])

        def wait_in(b):
            pltpu.make_async_copy(ex_hbm.at[pl.ds(0, B1)], exb[b], semr[b]).wait()
            pltpu.make_async_copy(dst_hbm.at[pl.ds(0, B1)], dv[0], semr[b]).wait()

        def drain_w(b):
            pltpu.make_async_copy(exw[b], den_sp.at[dv[0]], semw[b]).wait()

        def step(i, u):
            b = u % 2
            wait_in(b)

            @pl.when(i >= 2)
            def _():
                drain_w(b)

            def edge(e, ecarry):
                exw[b][e, pl.ds(0, H)] = exb[b][e, :]
                return ecarry

            lax.fori_loop(0, B1, edge, 0)
            pltpu.async_copy(exw[b], den_sp.at[dv[u % 4]], semw[b], add=True)
            nxt = i + 2

            @pl.when(nxt < NCH1)
            def _():
                issue(nxt, b, (u + 2) % 4)

        issue(0, 0, 0)
        issue(1, 1, 1)

        def gbody(g, carry):
            for u in range(4):
                step(g * 4 + u, u)
            return carry

        lax.fori_loop(0, NCH1 // 4, gbody, 0)

        for t in range(NCH1 - (NCH1 // 4) * 4):
            i = (NCH1 // 4) * 4 + t
            step(i, i % 4)
        drain_w(0)
        drain_w(1)

        plsc.subcore_barrier()

        def drows(t, carry):
            ch = t * NS + sid

            @pl.when(ch < NRCH)
            def _():
                r0 = ch * RB
                pltpu.sync_copy(den_sp.at[pl.ds(r0, RB)],
                                den_hbm.at[pl.ds(cid * N + r0, RB)])

            return carry

        lax.fori_loop(0, (NRCH + NS - 1) // NS, drows, 0)

    return k(exv, dst, zden)


def _sc_agg(vlo, vhi, exv, src, dst, den2, zacc):
    @functools.partial(
        pl.kernel,
        out_type=jax.ShapeDtypeStruct((NC * N, DH), jnp.float32),
        mesh=_sc_mesh(),
        scratch_types=[
            pltpu.VMEM((EW2,), jnp.int32),
            [pltpu.VMEM((B2, DH), jnp.float32)] * 3,
            [pltpu.VMEM((B2, H), jnp.float32)] * 3,
            [pltpu.VMEM((B2,), jnp.int32)] * 3,
            pltpu.VMEM_SHARED((N, DH), jnp.float32),
            [pltpu.SemaphoreType.DMA] * 3,
            [pltpu.SemaphoreType.DMA] * 3,
        ],
    )
    def k(vlo_hbm, vhi_hbm, ex_hbm, src_hbm, dst_hbm, den_hbm, zacc_hbm,
          out_hbm, srcall, vb, exb, dv, acc_sp, semr, semw):
        cid = lax.axis_index("c")
        sid = lax.axis_index("s")
        base0 = sid * EW2

        @pl.when(sid == 0)
        def _():
            pltpu.sync_copy(zacc_hbm, acc_sp)

        pltpu.sync_copy(src_hbm.at[pl.ds(base0, EW2)], srcall)
        plsc.subcore_barrier()

        def issue(i, b):
            base = base0 + i * B2
            idx = srcall.at[pl.ds(i * B2, B2)]

            @pl.when(cid == 0)
            def _():
                pltpu.async_copy(vlo_hbm.at[idx], vb[b], semr[b])

            @pl.when(cid == 1)
            def _():
                pltpu.async_copy(vhi_hbm.at[idx], vb[b], semr[b])

            pltpu.async_copy(ex_hbm.at[pl.ds(base, B2)], exb[b], semr[b])
            pltpu.async_copy(dst_hbm.at[pl.ds(base, B2)], dv[b], semr[b])

        def wait_in(b):
            pltpu.make_async_copy(vlo_hbm.at[srcall.at[pl.ds(0, B2)]], vb[b], semr[b]).wait()
            pltpu.make_async_copy(ex_hbm.at[pl.ds(0, B2)], exb[b], semr[b]).wait()
            pltpu.make_async_copy(dst_hbm.at[pl.ds(0, B2)], dv[0], semr[b]).wait()

        def drain_w(b):
            pltpu.make_async_copy(vb[b], acc_sp.at[dv[0]], semw[b]).wait()

        def compute(b):
            def edge(e2, ecarry):
                for o in range(2):
                    e = e2 * 2 + o
                    ev = exb[b][e, :]
                    for j in range(DH // H):
                        sl = pl.ds(j * H, H)
                        vb[b][e, sl] = vb[b][e, sl] * ev
                return ecarry

            lax.fori_loop(0, B2 // 2, edge, 0)

        def step(i, u):
            b = u % 3
            wait_in(b)
            compute(b)
            pltpu.async_copy(vb[b], acc_sp.at[dv[b]], semw[b], add=True)
            prev = (u + 2) % 3
            if u == 0:
                @pl.when(i >= 1)
                def _():
                    drain_w(prev)
            else:
                drain_w(prev)
            nxt = i + 2

            @pl.when(nxt < NCH2)
            def _():
                issue(nxt, (b + 2) % 3)

        issue(0, 0)
        issue(1, 1)

        def gbody(g, carry):
            for u in range(3):
                step(g * 3 + u, u)
            return carry

        lax.fori_loop(0, NCH2 // 3, gbody, 0)

        for t in range(NCH2 - (NCH2 // 3) * 3):
            i = (NCH2 // 3) * 3 + t
            step(i, i % 3)
        drain_w((NCH2 - 1) % 3)

        plsc.subcore_barrier()

        def rows(t, carry):
            ch = t * NS + sid

            @pl.when(ch < NRCH2)
            def _():
                r0 = ch * RB2
                pltpu.sync_copy(acc_sp.at[pl.ds(r0, RB2)], vb[0])
                pltpu.sync_copy(den_hbm.at[pl.ds(r0, RB2)], vb[1])
                pltpu.sync_copy(den_hbm.at[pl.ds(N + r0, RB2)], vb[2])

                def row(r, rcarry):
                    dsl = pl.ds(0, H)
                    rec = 1.0 / (vb[1][r, dsl] + vb[2][r, dsl] + 1e-16)
                    for j in range(DH // H):
                        sl = pl.ds(j * H, H)
                        vb[0][r, sl] = vb[0][r, sl] * rec
                    return rcarry

                lax.fori_loop(0, RB2, row, 0)
                pltpu.sync_copy(vb[0], out_hbm.at[pl.ds(cid * N + r0, RB2)])

            return carry

        lax.fori_loop(0, (NRCH2 + NS - 1) // NS, rows, 0)

    return k(vlo, vhi, exv, src, dst, den2, zacc)


def _perm_cols(w):
    shp = w.shape[:-1]
    return w.reshape(*shp, H, C).swapaxes(-1, -2).reshape(*shp, D)


def kernel(x, edge_index, edge_attr, ln1_g, ln1_b, Wq, bq, Wk, bk, Wv, bv,
           Ws, bs, We, be, Wp, bp, mlp_ln_g, mlp_ln_b, W1, b1, W2, b2):
    scale = 1.0 / (C ** 0.5)
    wq_p = _perm_cols(Wq) * scale
    bq_p = _perm_cols(bq) * scale
    wk_p = _perm_cols(Wk)
    bk_p = _perm_cols(bk)
    wv_p = _perm_cols(Wv)
    bv_p = _perm_cols(bv)
    ws_p = _perm_cols(Ws)
    bs_p = _perm_cols(bs)
    we_p = _perm_cols(We)
    be_p = _perm_cols(be)
    wp_p = Wp.reshape(H, C, D).swapaxes(0, 1).reshape(D, D)

    q, kk, vlo, vhi, xr = _pre(x, ln1_g, ln1_b, wq_p, bq_p, wk_p, bk_p,
                               wv_p, bv_p, ws_p, bs_p)
    eemb = _eemb(edge_attr, we_p, be_p)

    src = edge_index[0].astype(jnp.int32)
    dst = edge_index[1].astype(jnp.int32)

    exv = _sc_attn(q, kk, eemb, src, dst)
    zden = jnp.zeros((N, DW), jnp.float32)
    den2 = _sc_den(exv, dst, zden)

    zacc = jnp.zeros((N, DH), jnp.float32)
    out2 = _sc_agg(vlo, vhi, exv, src, dst, den2, zacc)

    return _tail(out2, xr, x, wp_p, bp, mlp_ln_g, mlp_ln_b, W1, b1, W2, b2)
